# Initial kernel scaffold; baseline (speedup 1.0000x reference)
import functools

import jax
import jax.numpy as jnp
from jax import lax
from jax.experimental import pallas as pl
from jax.experimental.pallas import tpu as pltpu
from jax.experimental.pallas import tpu_sc as plsc

_N = 10000
_E = 160000
_NT = 16
_NC = 2
_B = 128
_NB = 80
_NBT = _NT * _NB
_NPAD = 10016
_ZR = 313
_FLUSH = _N // _NT
_RB = 1000


def _deg_body(dstg, ones, zeros8, out, dstv, onesv, zbuf8, acc):
    c = lax.axis_index("c")
    t = lax.axis_index("s")
    pltpu.sync_copy(dstg.at[pl.ds(t * _NB, _NB)], dstv)
    pltpu.sync_copy(ones, onesv)
    pltpu.sync_copy(zeros8, zbuf8)
    pltpu.sync_copy(zbuf8, acc.at[pl.ds(t * (2 * _ZR), 2 * _ZR)])
    plsc.subcore_barrier()
    half = _NB // _NC

    def batch(j, carry):
        pltpu.sync_copy(onesv, acc.at[dstv.at[c * half + j]], add=True)
        return carry

    lax.fori_loop(0, half, batch, 0)
    plsc.subcore_barrier()
    pltpu.sync_copy(acc.at[pl.ds(t * _FLUSH, _FLUSH)],
                    out.at[pl.ds(c * _N + t * _FLUSH, _FLUSH)])


def _deg_kernel(dstg, ones, zeros8):
    mesh = plsc.VectorSubcoreMesh(core_axis_name="c", subcore_axis_name="s")
    return pl.kernel(
        _deg_body,
        out_type=jax.ShapeDtypeStruct((_NC * _N, 8), jnp.float32),
        mesh=mesh,
        scratch_types=[
            pltpu.VMEM((_NB, _B), jnp.int32),
            pltpu.VMEM((_B, 8), jnp.float32),
            pltpu.VMEM((2 * _ZR, 8), jnp.float32),
            pltpu.VMEM_SHARED((_NPAD, 8), jnp.float32),
        ],
    )(dstg, ones, zeros8)


def _make_scatter_body(C):
    CPC = C // _NC

    def body(table, srcg, dstg, zeros, out, srcv, dstv, buf0, buf1, zbuf,
             acc, sem0, sem1):
        c = lax.axis_index("c")
        t = lax.axis_index("s")
        pltpu.sync_copy(dstg.at[pl.ds(t * _NB, _NB)], dstv)
        pltpu.sync_copy(zeros, zbuf)
        for k in range(CPC):
            g = c * CPC + k
            pltpu.sync_copy(zbuf, acc.at[pl.ds(t * (2 * _ZR), _ZR)])
            pltpu.sync_copy(zbuf, acc.at[pl.ds(t * (2 * _ZR) + _ZR, _ZR)])
            pltpu.sync_copy(srcg.at[pl.ds((g * _NT + t) * _NB, _NB)], srcv)
            plsc.subcore_barrier()

            pltpu.async_copy(table.at[srcv.at[0]], buf0, sem0)

            def pair(jj, carry):
                j0 = 2 * jj
                pltpu.async_copy(table.at[srcv.at[j0 + 1]], buf1, sem1)
                pltpu.make_async_copy(table.at[srcv.at[j0]], buf0, sem0).wait()
                pltpu.sync_copy(buf0, acc.at[dstv.at[j0]], add=True)

                @pl.when(jj < _NB // 2 - 1)
                def _():
                    pltpu.async_copy(table.at[srcv.at[j0 + 2]], buf0, sem0)

                pltpu.make_async_copy(table.at[srcv.at[j0 + 1]], buf1,
                                      sem1).wait()
                pltpu.sync_copy(buf1, acc.at[dstv.at[j0 + 1]], add=True)
                return carry

            lax.fori_loop(0, _NB // 2, pair, 0)
            plsc.subcore_barrier()
            pltpu.sync_copy(acc.at[pl.ds(t * _FLUSH, _FLUSH)],
                            out.at[pl.ds(g * _N + t * _FLUSH, _FLUSH)])
            plsc.subcore_barrier()

    return body


def _make_scatter_kernel(C):
    mesh = plsc.VectorSubcoreMesh(core_axis_name="c", subcore_axis_name="s")
    return pl.kernel(
        _make_scatter_body(C),
        out_type=jax.ShapeDtypeStruct((C * _N, 128), jnp.float32),
        mesh=mesh,
        scratch_types=[
            pltpu.VMEM((_NB, _B), jnp.int32),
            pltpu.VMEM((_NB, _B), jnp.int32),
            pltpu.VMEM((_B, 128), jnp.float32),
            pltpu.VMEM((_B, 128), jnp.float32),
            pltpu.VMEM((_ZR, 128), jnp.float32),
            pltpu.VMEM_SHARED((_NPAD, 128), jnp.float32),
            pltpu.SemaphoreType.DMA,
            pltpu.SemaphoreType.DMA,
        ],
    )


_scatter4 = _make_scatter_kernel(4)
_scatter2 = _make_scatter_kernel(2)


def _dinv(dp_ref):
    return lax.rsqrt(dp_ref[:, 0:1] + dp_ref[:, 1:2] + 1.0)


def _k1_body(x_ref, w_ref, dp_ref, o_ref):
    h = jnp.dot(x_ref[...], w_ref[...], preferred_element_type=jnp.float32)
    h = h * _dinv(dp_ref)
    for cc in range(4):
        o_ref[cc] = h[:, cc * 128:(cc + 1) * 128]


def _mid_body(cin, cout, s_ref, h_ref, dp_ref, b_ref, w_ref, o_ref):
    dinv = _dinv(dp_ref)
    s = jnp.concatenate([s_ref[cc] for cc in range(cin)], axis=1)
    hp = jnp.concatenate([h_ref[cc] for cc in range(cin)], axis=1)
    xn = jnp.maximum(dinv * (s + hp) + b_ref[...], 0.0)
    h2 = jnp.dot(xn, w_ref[...], preferred_element_type=jnp.float32) * dinv
    for cc in range(cout):
        o_ref[cc] = h2[:, cc * 128:(cc + 1) * 128]


def _k4_body(s_ref, h_ref, dp_ref, b3_ref, x_ref, m1a_ref, m1b_ref,
             m1bias_ref, m2w_ref, m2b_ref, m3w_ref, m3b_ref, o_ref):
    dinv = _dinv(dp_ref)
    s = jnp.concatenate([s_ref[cc] for cc in range(2)], axis=1)
    hp = jnp.concatenate([h_ref[cc] for cc in range(2)], axis=1)
    emb = dinv * (s + hp) + b3_ref[...]
    z = jnp.dot(x_ref[...], m1a_ref[...], preferred_element_type=jnp.float32)
    z = z + jnp.dot(emb, m1b_ref[...], preferred_element_type=jnp.float32)
    z = jnp.maximum(z + m1bias_ref[...], 0.0)
    z = jnp.maximum(
        jnp.dot(z, m2w_ref[...], preferred_element_type=jnp.float32)
        + m2b_ref[...], 0.0)
    o_ref[...] = (jnp.dot(z, m3w_ref[...], preferred_element_type=jnp.float32)
                  + m3b_ref[...])


def _row_spec(shape3):
    return pl.BlockSpec(shape3, lambda i: (0, i, 0))


_DP_SPEC = pl.BlockSpec((_RB, 2), lambda i: (i, 0))


def _full_spec(shape):
    nd = len(shape)
    return pl.BlockSpec(shape, lambda i, _n=nd: (0,) * _n)


def _k1(x, w1, dp):
    return pl.pallas_call(
        _k1_body,
        grid=(_N // _RB,),
        in_specs=[
            pl.BlockSpec((_RB, 256), lambda i: (i, 0)),
            _full_spec((256, 512)),
            _DP_SPEC,
        ],
        out_specs=_row_spec((4, _RB, 128)),
        out_shape=jax.ShapeDtypeStruct((4, _N, 128), jnp.float32),
    )(x, w1, dp)


def _k_mid(cin, cout, s, h, dp, b, w):
    din, dout = cin * 128, cout * 128
    return pl.pallas_call(
        functools.partial(_mid_body, cin, cout),
        grid=(_N // _RB,),
        in_specs=[
            _row_spec((cin, _RB, 128)),
            _row_spec((cin, _RB, 128)),
            _DP_SPEC,
            _full_spec((1, din)),
            _full_spec((din, dout)),
        ],
        out_specs=_row_spec((cout, _RB, 128)),
        out_shape=jax.ShapeDtypeStruct((cout, _N, 128), jnp.float32),
    )(s, h, dp, b, w)


def _k4(s3, h3, dp, b3, x, m1a, m1b, m1bias, m2w, m2b, m3w, m3b):
    return pl.pallas_call(
        _k4_body,
        grid=(_N // _RB,),
        in_specs=[
            _row_spec((2, _RB, 128)),
            _row_spec((2, _RB, 128)),
            _DP_SPEC,
            _full_spec((1, 256)),
            pl.BlockSpec((_RB, 256), lambda i: (i, 0)),
            _full_spec((256, 512)),
            _full_spec((256, 512)),
            _full_spec((1, 512)),
            _full_spec((512, 256)),
            _full_spec((1, 256)),
            _full_spec((256, 1)),
            _full_spec((1, 1)),
        ],
        out_specs=pl.BlockSpec((_RB, 1), lambda i: (i, 0)),
        out_shape=jax.ShapeDtypeStruct((_N, 1), jnp.float32),
    )(s3, h3, dp, b3, x, m1a, m1b, m1bias, m2w, m2b, m3w, m3b)


def kernel(x, edge_indices, W1, b1, W2, b2, W3, b3,
           M1w, M1b, M2w, M2b, M3w, M3b):
    src = edge_indices[0].astype(jnp.int32)
    dst = edge_indices[1].astype(jnp.int32)

    padn = _NBT * _B - _E
    srcp = jnp.concatenate([src, jnp.zeros((padn,), jnp.int32)])
    dstp = jnp.concatenate([dst, jnp.full((padn,), _N, jnp.int32)])
    srcp = srcp.reshape(_NBT, _B)
    dstg = dstp.reshape(_NBT, _B)
    srcg4 = (srcp[None] + (jnp.arange(4, dtype=jnp.int32) * _N)[:, None, None]
             ).reshape(4 * _NBT, _B)
    srcg2 = (srcp[None] + (jnp.arange(2, dtype=jnp.int32) * _N)[:, None, None]
             ).reshape(2 * _NBT, _B)

    ones = jnp.zeros((_B, 8), jnp.float32).at[:, 0].set(1.0)
    zeros8 = jnp.zeros((2 * _ZR, 8), jnp.float32)
    zeros = jnp.zeros((_ZR, 128), jnp.float32)

    degp = _deg_kernel(dstg, ones, zeros8)
    dp = jnp.stack([degp[:_N, 0], degp[_N:, 0]], axis=1)

    h1 = _k1(x, W1, dp)
    s1 = _scatter4(h1.reshape(4 * _N, 128), srcg4, dstg, zeros)
    h2 = _k_mid(4, 4, s1.reshape(4, _N, 128), h1, dp, b1.reshape(1, -1), W2)
    s2 = _scatter4(h2.reshape(4 * _N, 128), srcg4, dstg, zeros)
    h3 = _k_mid(4, 2, s2.reshape(4, _N, 128), h2, dp, b2.reshape(1, -1), W3)
    s3 = _scatter2(h3.reshape(2 * _N, 128), srcg2, dstg, zeros)
    out = _k4(s3.reshape(2, _N, 128), h3, dp, b3.reshape(1, -1), x,
              M1w[:256], M1w[256:], M1b.reshape(1, -1),
              M2w, M2b.reshape(1, -1), M3w, M3b.reshape(1, 1))
    return out


# speedup vs baseline: 5.6165x; 5.6165x over previous
import functools

import jax
import jax.numpy as jnp
from jax import lax
from jax.experimental import pallas as pl
from jax.experimental.pallas import tpu as pltpu
from jax.experimental.pallas import tpu_sc as plsc

_N = 10000
_E = 160000
_NT = 16
_NC = 2
_B = 128
_NB = 80
_NBT = _NT * _NB
_NPAD = 10240
_ZR = 128
_CW = 64
_FLUSH = 624
_RB = 1000


def _flush(acc, out, t, base):
    pltpu.sync_copy(acc.at[pl.ds(t * _FLUSH, _FLUSH)],
                    out.at[pl.ds(base + t * _FLUSH, _FLUSH)])

    @pl.when(t == _NT - 1)
    def _():
        pltpu.sync_copy(acc.at[pl.ds(_NT * _FLUSH, _N - _NT * _FLUSH)],
                        out.at[pl.ds(base + _NT * _FLUSH,
                                     _N - _NT * _FLUSH)])


def _deg_body(dstg, ones, zeros8, out, dstv, onesv, zbuf8, acc):
    c = lax.axis_index("c")
    t = lax.axis_index("s")
    pltpu.sync_copy(dstg.at[pl.ds(t * _NB, _NB)], dstv)
    pltpu.sync_copy(ones, onesv)
    pltpu.sync_copy(zeros8, zbuf8)
    pltpu.sync_copy(zbuf8, acc.at[pl.ds(t * (_NPAD // _NT), _NPAD // _NT)])
    plsc.subcore_barrier()
    half = _NB // _NC

    def batch(j, carry):
        pltpu.sync_copy(onesv, acc.at[dstv.at[c * half + j]], add=True)
        return carry

    lax.fori_loop(0, half, batch, 0)
    plsc.subcore_barrier()
    _flush(acc, out, t, c * _N)


def _deg_kernel(dstg, ones, zeros8):
    mesh = plsc.VectorSubcoreMesh(core_axis_name="c", subcore_axis_name="s")
    return pl.kernel(
        _deg_body,
        out_type=jax.ShapeDtypeStruct((_NC * _N, 8), jnp.float32),
        mesh=mesh,
        compiler_params=pltpu.CompilerParams(use_tc_tiling_on_sc=False),
        scratch_types=[
            pltpu.VMEM((_NB, _B), jnp.int32),
            pltpu.VMEM((_B, 8), jnp.float32),
            pltpu.VMEM((_NPAD // _NT, 8), jnp.float32),
            pltpu.VMEM_SHARED((_NPAD, 8), jnp.float32),
        ],
    )(dstg, ones, zeros8)


def _make_scatter_body(C):
    CPC = C // _NC

    def body(table, srcg, dstg, zeros, out, srcv, dstv, buf0, buf1, zbuf,
             acc, sem0, sem1):
        c = lax.axis_index("c")
        t = lax.axis_index("s")
        pltpu.sync_copy(dstg.at[pl.ds(t * _NB, _NB)], dstv)
        pltpu.sync_copy(zeros, zbuf)
        for k in range(CPC):
            g = c * CPC + k
            for z in range(5):
                pltpu.sync_copy(zbuf,
                                acc.at[pl.ds(t * (_NPAD // _NT) + z * _ZR,
                                             _ZR)])
            pltpu.sync_copy(srcg.at[pl.ds((g * _NT + t) * _NB, _NB)], srcv)
            plsc.subcore_barrier()

            pltpu.async_copy(table.at[srcv.at[0]], buf0, sem0)

            def pair(jj, carry):
                j0 = 2 * jj
                pltpu.async_copy(table.at[srcv.at[j0 + 1]], buf1, sem1)
                pltpu.make_async_copy(table.at[srcv.at[j0]], buf0, sem0).wait()
                pltpu.sync_copy(buf0, acc.at[dstv.at[j0]], add=True)

                @pl.when(jj < _NB // 2 - 1)
                def _():
                    pltpu.async_copy(table.at[srcv.at[j0 + 2]], buf0, sem0)

                pltpu.make_async_copy(table.at[srcv.at[j0 + 1]], buf1,
                                      sem1).wait()
                pltpu.sync_copy(buf1, acc.at[dstv.at[j0 + 1]], add=True)
                return carry

            lax.fori_loop(0, _NB // 2, pair, 0)
            plsc.subcore_barrier()
            _flush(acc, out, t, g * _N)
            plsc.subcore_barrier()

    return body


@functools.lru_cache(maxsize=None)
def _make_scatter_kernel(C):
    mesh = plsc.VectorSubcoreMesh(core_axis_name="c", subcore_axis_name="s")
    return pl.kernel(
        _make_scatter_body(C),
        out_type=jax.ShapeDtypeStruct((C * _N, _CW), jnp.float32),
        mesh=mesh,
        compiler_params=pltpu.CompilerParams(use_tc_tiling_on_sc=False),
        scratch_types=[
            pltpu.VMEM((_NB, _B), jnp.int32),
            pltpu.VMEM((_NB, _B), jnp.int32),
            pltpu.VMEM((_B, _CW), jnp.float32),
            pltpu.VMEM((_B, _CW), jnp.float32),
            pltpu.VMEM((_ZR, _CW), jnp.float32),
            pltpu.VMEM_SHARED((_NPAD, _CW), jnp.float32),
            pltpu.SemaphoreType.DMA,
            pltpu.SemaphoreType.DMA,
        ],
    )


def _scatter8(*args):
    return _make_scatter_kernel(8)(*args)


def _scatter4(*args):
    return _make_scatter_kernel(4)(*args)


def _dinv(dp_ref):
    return lax.rsqrt(dp_ref[:, 0:1] + dp_ref[:, 1:2] + 1.0)


def _k1_body(x_ref, w_ref, dp_ref, o_ref):
    h = jnp.dot(x_ref[...], w_ref[...], preferred_element_type=jnp.float32)
    h = h * _dinv(dp_ref)
    for cc in range(8):
        o_ref[cc] = h[:, cc * _CW:(cc + 1) * _CW]


def _mid_body(cin, cout, s_ref, h_ref, dp_ref, b_ref, w_ref, o_ref):
    dinv = _dinv(dp_ref)
    s = jnp.concatenate([s_ref[cc] for cc in range(cin)], axis=1)
    hp = jnp.concatenate([h_ref[cc] for cc in range(cin)], axis=1)
    xn = jnp.maximum(dinv * (s + hp) + b_ref[...], 0.0)
    h2 = jnp.dot(xn, w_ref[...], preferred_element_type=jnp.float32) * dinv
    for cc in range(cout):
        o_ref[cc] = h2[:, cc * _CW:(cc + 1) * _CW]


def _k4_body(s_ref, h_ref, dp_ref, b3_ref, x_ref, m1a_ref, m1b_ref,
             m1bias_ref, m2w_ref, m2b_ref, m3w_ref, m3b_ref, o_ref):
    dinv = _dinv(dp_ref)
    s = jnp.concatenate([s_ref[cc] for cc in range(4)], axis=1)
    hp = jnp.concatenate([h_ref[cc] for cc in range(4)], axis=1)
    emb = dinv * (s + hp) + b3_ref[...]
    z = jnp.dot(x_ref[...], m1a_ref[...], preferred_element_type=jnp.float32)
    z = z + jnp.dot(emb, m1b_ref[...], preferred_element_type=jnp.float32)
    z = jnp.maximum(z + m1bias_ref[...], 0.0)
    z = jnp.maximum(
        jnp.dot(z, m2w_ref[...], preferred_element_type=jnp.float32)
        + m2b_ref[...], 0.0)
    o_ref[...] = (jnp.dot(z, m3w_ref[...], preferred_element_type=jnp.float32)
                  + m3b_ref[...])


def _row_spec(shape3):
    return pl.BlockSpec(shape3, lambda i: (0, i, 0))


_DP_SPEC = pl.BlockSpec((_RB, 2), lambda i: (i, 0))


def _full_spec(shape):
    nd = len(shape)
    return pl.BlockSpec(shape, lambda i, _n=nd: (0,) * _n)


def _k1(x, w1, dp):
    return pl.pallas_call(
        _k1_body,
        grid=(_N // _RB,),
        in_specs=[
            pl.BlockSpec((_RB, 256), lambda i: (i, 0)),
            _full_spec((256, 512)),
            _DP_SPEC,
        ],
        out_specs=_row_spec((8, _RB, _CW)),
        out_shape=jax.ShapeDtypeStruct((8, _N, _CW), jnp.float32),
    )(x, w1, dp)


def _k_mid(cin, cout, s, h, dp, b, w):
    din, dout = cin * _CW, cout * _CW
    return pl.pallas_call(
        functools.partial(_mid_body, cin, cout),
        grid=(_N // _RB,),
        in_specs=[
            _row_spec((cin, _RB, _CW)),
            _row_spec((cin, _RB, _CW)),
            _DP_SPEC,
            _full_spec((1, din)),
            _full_spec((din, dout)),
        ],
        out_specs=_row_spec((cout, _RB, _CW)),
        out_shape=jax.ShapeDtypeStruct((cout, _N, _CW), jnp.float32),
    )(s, h, dp, b, w)


def _k4(s3, h3, dp, b3, x, m1a, m1b, m1bias, m2w, m2b, m3w, m3b):
    return pl.pallas_call(
        _k4_body,
        grid=(_N // _RB,),
        in_specs=[
            _row_spec((4, _RB, _CW)),
            _row_spec((4, _RB, _CW)),
            _DP_SPEC,
            _full_spec((1, 256)),
            pl.BlockSpec((_RB, 256), lambda i: (i, 0)),
            _full_spec((256, 512)),
            _full_spec((256, 512)),
            _full_spec((1, 512)),
            _full_spec((512, 256)),
            _full_spec((1, 256)),
            _full_spec((256, 1)),
            _full_spec((1, 1)),
        ],
        out_specs=pl.BlockSpec((_RB, 1), lambda i: (i, 0)),
        out_shape=jax.ShapeDtypeStruct((_N, 1), jnp.float32),
    )(s3, h3, dp, b3, x, m1a, m1b, m1bias, m2w, m2b, m3w, m3b)


def kernel(x, edge_indices, W1, b1, W2, b2, W3, b3,
           M1w, M1b, M2w, M2b, M3w, M3b):
    src = edge_indices[0].astype(jnp.int32)
    dst = edge_indices[1].astype(jnp.int32)

    padn = _NBT * _B - _E
    srcp = jnp.concatenate([src, jnp.zeros((padn,), jnp.int32)])
    dstp = jnp.concatenate([dst, jnp.full((padn,), _N, jnp.int32)])
    srcp = srcp.reshape(_NBT, _B)
    dstg = dstp.reshape(_NBT, _B)
    srcg8 = (srcp[None] + (jnp.arange(8, dtype=jnp.int32) * _N)[:, None, None]
             ).reshape(8 * _NBT, _B)
    srcg4 = (srcp[None] + (jnp.arange(4, dtype=jnp.int32) * _N)[:, None, None]
             ).reshape(4 * _NBT, _B)

    ones = jnp.zeros((_B, 8), jnp.float32).at[:, 0].set(1.0)
    zeros8 = jnp.zeros((_NPAD // _NT, 8), jnp.float32)
    zeros = jnp.zeros((_ZR, _CW), jnp.float32)

    degp = _deg_kernel(dstg, ones, zeros8)
    dp = jnp.stack([degp[:_N, 0], degp[_N:, 0]], axis=1)

    h1 = _k1(x, W1, dp)
    s1 = _scatter8(h1.reshape(8 * _N, _CW), srcg8, dstg, zeros)
    h2 = _k_mid(8, 8, s1.reshape(8, _N, _CW), h1, dp, b1.reshape(1, -1), W2)
    s2 = _scatter8(h2.reshape(8 * _N, _CW), srcg8, dstg, zeros)
    h3 = _k_mid(8, 4, s2.reshape(8, _N, _CW), h2, dp, b2.reshape(1, -1), W3)
    s3 = _scatter4(h3.reshape(4 * _N, _CW), srcg4, dstg, zeros)
    out = _k4(s3.reshape(4, _N, _CW), h3, dp, b3.reshape(1, -1), x,
              M1w[:256], M1w[256:], M1b.reshape(1, -1),
              M2w, M2b.reshape(1, -1), M3w, M3b.reshape(1, 1))
    return out


# speedup vs baseline: 5.6766x; 1.0107x over previous
import functools

import jax
import jax.numpy as jnp
from jax import lax
from jax.experimental import pallas as pl
from jax.experimental.pallas import tpu as pltpu
from jax.experimental.pallas import tpu_sc as plsc

_N = 10000
_E = 160000
_NT = 16
_NC = 2
_B = 128
_NB = 80
_NBT = _NT * _NB
_NPAD = 10240
_ZR = 128
_CW = 64
_FLUSH = 624
_RB = 1000


def _flush(acc, out, t, base):
    pltpu.sync_copy(acc.at[pl.ds(t * _FLUSH, _FLUSH)],
                    out.at[pl.ds(base + t * _FLUSH, _FLUSH)])

    @pl.when(t == _NT - 1)
    def _():
        pltpu.sync_copy(acc.at[pl.ds(_NT * _FLUSH, _N - _NT * _FLUSH)],
                        out.at[pl.ds(base + _NT * _FLUSH,
                                     _N - _NT * _FLUSH)])


def _deg_body(dstg, ones, zeros8, out, dstv, onesv, zbuf8, acc):
    c = lax.axis_index("c")
    t = lax.axis_index("s")
    pltpu.sync_copy(dstg.at[pl.ds(t * _NB, _NB)], dstv)
    pltpu.sync_copy(ones, onesv)
    pltpu.sync_copy(zeros8, zbuf8)
    pltpu.sync_copy(zbuf8, acc.at[pl.ds(t * (_NPAD // _NT), _NPAD // _NT)])
    plsc.subcore_barrier()
    half = _NB // _NC

    def batch(j, carry):
        pltpu.sync_copy(onesv, acc.at[dstv.at[c * half + j]], add=True)
        return carry

    lax.fori_loop(0, half, batch, 0)
    plsc.subcore_barrier()
    _flush(acc, out, t, c * _N)


def _deg_kernel(dstg, ones, zeros8):
    mesh = plsc.VectorSubcoreMesh(core_axis_name="c", subcore_axis_name="s")
    return pl.kernel(
        _deg_body,
        out_type=jax.ShapeDtypeStruct((_NC * _N, 8), jnp.float32),
        mesh=mesh,
        compiler_params=pltpu.CompilerParams(use_tc_tiling_on_sc=False),
        scratch_types=[
            pltpu.VMEM((_NB, _B), jnp.int32),
            pltpu.VMEM((_B, 8), jnp.float32),
            pltpu.VMEM((_NPAD // _NT, 8), jnp.float32),
            pltpu.VMEM_SHARED((_NPAD, 8), jnp.float32),
        ],
    )(dstg, ones, zeros8)


def _make_scatter_body(C):
    CPC = C // _NC
    NRING = 4

    def body(table, srcg, dstg, zeros, out, srcv, dstv, bufs, zbuf,
             acc, gsems, ssems):
        c = lax.axis_index("c")
        t = lax.axis_index("s")
        pltpu.sync_copy(dstg.at[pl.ds(t * _NB, _NB)], dstv)
        pltpu.sync_copy(srcg.at[pl.ds(t * _NB, _NB)], srcv)
        pltpu.sync_copy(zeros, zbuf)
        for k in range(CPC):
            g = c * CPC + k
            tbl = table.at[pl.ds(g * _N, _N)]
            for z in range(5):
                pltpu.sync_copy(zbuf,
                                acc.at[pl.ds(t * (_NPAD // _NT) + z * _ZR,
                                             _ZR)])
            plsc.subcore_barrier()

            def gath_start(j, r):
                pltpu.async_copy(tbl.at[srcv.at[j]], bufs.at[r], gsems[r])

            def gath_wait(j, r):
                pltpu.make_async_copy(tbl.at[srcv.at[j]], bufs.at[r],
                                      gsems[r]).wait()

            def scat_start(j, r):
                pltpu.async_copy(bufs.at[r], acc.at[dstv.at[j]], ssems[r],
                                 add=True)

            def scat_wait(j, r):
                pltpu.make_async_copy(bufs.at[r], acc.at[dstv.at[j]],
                                      ssems[r]).wait()

            gath_start(0, 0)
            gath_start(1, 1)

            def ring(ii, carry):
                for r4 in range(NRING):
                    jj = NRING * ii + r4
                    rg = (r4 + 2) % NRING

                    @pl.when(jj >= 2)
                    def _():
                        scat_wait(jj - 2, rg)

                    @pl.when(jj + 2 < _NB)
                    def _():
                        gath_start(jj + 2, rg)

                    gath_wait(jj, r4)
                    scat_start(jj, r4)
                return carry

            lax.fori_loop(0, _NB // NRING, ring, 0)
            scat_wait(_NB - 2, (_NB - 2) % NRING)
            scat_wait(_NB - 1, (_NB - 1) % NRING)
            plsc.subcore_barrier()
            _flush(acc, out, t, g * _N)
            plsc.subcore_barrier()

    return body


@functools.lru_cache(maxsize=None)
def _make_scatter_kernel(C):
    mesh = plsc.VectorSubcoreMesh(core_axis_name="c", subcore_axis_name="s")
    return pl.kernel(
        _make_scatter_body(C),
        out_type=jax.ShapeDtypeStruct((C * _N, _CW), jnp.float32),
        mesh=mesh,
        compiler_params=pltpu.CompilerParams(use_tc_tiling_on_sc=False),
        scratch_types=[
            pltpu.VMEM((_NB, _B), jnp.int32),
            pltpu.VMEM((_NB, _B), jnp.int32),
            pltpu.VMEM((4, _B, _CW), jnp.float32),
            pltpu.VMEM((_ZR, _CW), jnp.float32),
            pltpu.VMEM_SHARED((_NPAD, _CW), jnp.float32),
            [pltpu.SemaphoreType.DMA] * 4,
            [pltpu.SemaphoreType.DMA] * 4,
        ],
    )


def _scatter8(*args):
    return _make_scatter_kernel(8)(*args)


def _scatter4(*args):
    return _make_scatter_kernel(4)(*args)


def _dinv(dp_ref):
    return lax.rsqrt(dp_ref[:, 0:1] + dp_ref[:, 1:2] + 1.0)


def _k1_body(x_ref, w_ref, dp_ref, o_ref):
    h = jnp.dot(x_ref[...], w_ref[...], preferred_element_type=jnp.float32)
    h = h * _dinv(dp_ref)
    for cc in range(8):
        o_ref[cc] = h[:, cc * _CW:(cc + 1) * _CW]


def _mid_body(cin, cout, s_ref, h_ref, dp_ref, b_ref, w_ref, o_ref):
    dinv = _dinv(dp_ref)
    s = jnp.concatenate([s_ref[cc] for cc in range(cin)], axis=1)
    hp = jnp.concatenate([h_ref[cc] for cc in range(cin)], axis=1)
    xn = jnp.maximum(dinv * (s + hp) + b_ref[...], 0.0)
    h2 = jnp.dot(xn, w_ref[...], preferred_element_type=jnp.float32) * dinv
    for cc in range(cout):
        o_ref[cc] = h2[:, cc * _CW:(cc + 1) * _CW]


def _k4_body(s_ref, h_ref, dp_ref, b3_ref, x_ref, m1a_ref, m1b_ref,
             m1bias_ref, m2w_ref, m2b_ref, m3w_ref, m3b_ref, o_ref):
    dinv = _dinv(dp_ref)
    s = jnp.concatenate([s_ref[cc] for cc in range(4)], axis=1)
    hp = jnp.concatenate([h_ref[cc] for cc in range(4)], axis=1)
    emb = dinv * (s + hp) + b3_ref[...]
    z = jnp.dot(x_ref[...], m1a_ref[...], preferred_element_type=jnp.float32)
    z = z + jnp.dot(emb, m1b_ref[...], preferred_element_type=jnp.float32)
    z = jnp.maximum(z + m1bias_ref[...], 0.0)
    z = jnp.maximum(
        jnp.dot(z, m2w_ref[...], preferred_element_type=jnp.float32)
        + m2b_ref[...], 0.0)
    o_ref[...] = (jnp.dot(z, m3w_ref[...], preferred_element_type=jnp.float32)
                  + m3b_ref[...])


def _row_spec(shape3):
    return pl.BlockSpec(shape3, lambda i: (0, i, 0))


_DP_SPEC = pl.BlockSpec((_RB, 2), lambda i: (i, 0))


def _full_spec(shape):
    nd = len(shape)
    return pl.BlockSpec(shape, lambda i, _n=nd: (0,) * _n)


def _k1(x, w1, dp):
    return pl.pallas_call(
        _k1_body,
        grid=(_N // _RB,),
        in_specs=[
            pl.BlockSpec((_RB, 256), lambda i: (i, 0)),
            _full_spec((256, 512)),
            _DP_SPEC,
        ],
        out_specs=_row_spec((8, _RB, _CW)),
        out_shape=jax.ShapeDtypeStruct((8, _N, _CW), jnp.float32),
    )(x, w1, dp)


def _k_mid(cin, cout, s, h, dp, b, w):
    din, dout = cin * _CW, cout * _CW
    return pl.pallas_call(
        functools.partial(_mid_body, cin, cout),
        grid=(_N // _RB,),
        in_specs=[
            _row_spec((cin, _RB, _CW)),
            _row_spec((cin, _RB, _CW)),
            _DP_SPEC,
            _full_spec((1, din)),
            _full_spec((din, dout)),
        ],
        out_specs=_row_spec((cout, _RB, _CW)),
        out_shape=jax.ShapeDtypeStruct((cout, _N, _CW), jnp.float32),
    )(s, h, dp, b, w)


def _k4(s3, h3, dp, b3, x, m1a, m1b, m1bias, m2w, m2b, m3w, m3b):
    return pl.pallas_call(
        _k4_body,
        grid=(_N // _RB,),
        in_specs=[
            _row_spec((4, _RB, _CW)),
            _row_spec((4, _RB, _CW)),
            _DP_SPEC,
            _full_spec((1, 256)),
            pl.BlockSpec((_RB, 256), lambda i: (i, 0)),
            _full_spec((256, 512)),
            _full_spec((256, 512)),
            _full_spec((1, 512)),
            _full_spec((512, 256)),
            _full_spec((1, 256)),
            _full_spec((256, 1)),
            _full_spec((1, 1)),
        ],
        out_specs=pl.BlockSpec((_RB, 1), lambda i: (i, 0)),
        out_shape=jax.ShapeDtypeStruct((_N, 1), jnp.float32),
    )(s3, h3, dp, b3, x, m1a, m1b, m1bias, m2w, m2b, m3w, m3b)


def kernel(x, edge_indices, W1, b1, W2, b2, W3, b3,
           M1w, M1b, M2w, M2b, M3w, M3b):
    src = edge_indices[0].astype(jnp.int32)
    dst = edge_indices[1].astype(jnp.int32)

    padn = _NBT * _B - _E
    srcp = jnp.concatenate([src, jnp.zeros((padn,), jnp.int32)])
    dstp = jnp.concatenate([dst, jnp.full((padn,), _N, jnp.int32)])
    srcg = srcp.reshape(_NBT, _B)
    dstg = dstp.reshape(_NBT, _B)


    ones = jnp.zeros((_B, 8), jnp.float32).at[:, 0].set(1.0)
    zeros8 = jnp.zeros((_NPAD // _NT, 8), jnp.float32)
    zeros = jnp.zeros((_ZR, _CW), jnp.float32)

    degp = _deg_kernel(dstg, ones, zeros8)
    dp = jnp.stack([degp[:_N, 0], degp[_N:, 0]], axis=1)

    h1 = _k1(x, W1, dp)
    s1 = _scatter8(h1.reshape(8 * _N, _CW), srcg, dstg, zeros)
    h2 = _k_mid(8, 8, s1.reshape(8, _N, _CW), h1, dp, b1.reshape(1, -1), W2)
    s2 = _scatter8(h2.reshape(8 * _N, _CW), srcg, dstg, zeros)
    h3 = _k_mid(8, 4, s2.reshape(8, _N, _CW), h2, dp, b2.reshape(1, -1), W3)
    s3 = _scatter4(h3.reshape(4 * _N, _CW), srcg, dstg, zeros)
    out = _k4(s3.reshape(4, _N, _CW), h3, dp, b3.reshape(1, -1), x,
              M1w[:256], M1w[256:], M1b.reshape(1, -1),
              M2w, M2b.reshape(1, -1), M3w, M3b.reshape(1, 1))
    return out
